# Initial kernel scaffold; baseline (speedup 1.0000x reference)
import jax
import jax.numpy as jnp
from jax.experimental import pallas as pl


def kernel(x, params):
    raise NotImplementedError("write your pallas kernel here")


# speedup vs baseline: 6.5456x; 6.5456x over previous
import functools

import jax
import jax.numpy as jnp
from jax import lax
from jax.experimental import pallas as pl
from jax.experimental.pallas import tpu as pltpu

L = 2048
DIM = 1024
DIN = 2048
DSTATE = 16
DTRANK = 64
DCONV = 4
NEXP = 8
FFI = 2048
LC = 64
RC = 256

_F32 = jnp.float32


def _silu(v):
    return v * jax.nn.sigmoid(v)


def _softplus(v):
    return jnp.maximum(v, 0.0) + jnp.log1p(jnp.exp(-jnp.abs(v)))


def _mamba_body(x_ref, w_in_ref, conv_w_ref, conv_b_ref, wd_ref, wb_ref,
                wc_ref, w_dt_ref, b_dt_ref, alog_ref, dd_ref, w_out_ref,
                out_ref, tail_ref, state_ref, da_s, dbu_s, st_s):
    c = pl.program_id(0)

    @pl.when(c == 0)
    def _():
        tail_ref[...] = jnp.zeros_like(tail_ref)
        state_ref[...] = jnp.zeros_like(state_ref)

    xch = x_ref[...]
    xz = jnp.dot(xch, w_in_ref[...], preferred_element_type=_F32)
    xc = xz[:, :DIN]
    res = xz[:, DIN:]

    ext = jnp.concatenate([tail_ref[...], xc], axis=0)
    tail_ref[...] = xc[LC - (DCONV - 1):, :]
    conv = conv_b_ref[...]
    for k in range(DCONV):
        conv = conv + ext[k:k + LC, :] * conv_w_ref[k:k + 1, :]
    xcs = _silu(conv)

    delta_r = jnp.dot(xcs, wd_ref[...], preferred_element_type=_F32)
    bm = jnp.dot(xcs, wb_ref[...], preferred_element_type=_F32)
    cm = jnp.dot(xcs, wc_ref[...], preferred_element_type=_F32)
    delta = _softplus(
        jnp.dot(delta_r, w_dt_ref[...], preferred_element_type=_F32)
        + b_dt_ref[...])
    u = delta * xcs

    a2 = -jnp.exp(alog_ref[...])
    da_s[...] = jnp.exp(delta[:, None, :] * a2[None, :, :])
    dbu_s[...] = bm[:, :, None] * u[:, None, :]

    def step(l, _):
        st = (da_s[pl.ds(l, 1)][0] * state_ref[...]
              + dbu_s[pl.ds(l, 1)][0])
        state_ref[...] = st
        st_s[pl.ds(l, 1)] = st[None]
        return 0

    lax.fori_loop(0, LC, step, 0, unroll=False)

    y = jnp.sum(st_s[...] * cm[:, :, None], axis=1)
    y = y + xcs * dd_ref[...]
    y = y * _silu(res)
    out_ref[...] = jnp.dot(y, w_out_ref[...], preferred_element_type=_F32) + xch


def _mamba_block(h, bp):
    w_in_t = bp['W_in'].T
    conv_w_t = bp['conv_w'].T
    conv_b = bp['conv_b'].reshape(1, DIN)
    wx = bp['W_x']
    wd_t = wx[:DTRANK].T
    wb_t = wx[DTRANK:DTRANK + DSTATE].T
    wc_t = wx[DTRANK + DSTATE:].T
    w_dt_t = bp['W_dt'].T
    b_dt = bp['b_dt'].reshape(1, DIN)
    alog_t = bp['A_log'].T
    dd = bp['D'].reshape(1, DIN)
    w_out_t = bp['W_out'].T

    grid = L // LC
    full = lambda shape: pl.BlockSpec(shape, lambda c: (0,) * len(shape))
    return pl.pallas_call(
        _mamba_body,
        grid=(grid,),
        in_specs=[
            pl.BlockSpec((LC, DIM), lambda c: (c, 0)),
            full((DIM, 2 * DIN)),
            full((DCONV, DIN)),
            full((1, DIN)),
            full((DIN, DTRANK)),
            full((DIN, DSTATE)),
            full((DIN, DSTATE)),
            full((DTRANK, DIN)),
            full((1, DIN)),
            full((DSTATE, DIN)),
            full((1, DIN)),
            full((DIN, DIM)),
        ],
        out_specs=pl.BlockSpec((LC, DIM), lambda c: (c, 0)),
        out_shape=jax.ShapeDtypeStruct((L, DIM), _F32),
        scratch_shapes=[
            pltpu.VMEM((DCONV - 1, DIN), _F32),
            pltpu.VMEM((DSTATE, DIN), _F32),
            pltpu.VMEM((LC, DSTATE, DIN), _F32),
            pltpu.VMEM((LC, DSTATE, DIN), _F32),
            pltpu.VMEM((LC, DSTATE, DIN), _F32),
        ],
    )(h, w_in_t, conv_w_t, conv_b, wd_t, wb_t, wc_t, w_dt_t, b_dt,
      alog_t, dd, w_out_t)


def _moe_body(h_ref, wg_ref, wgate_s_ref, wup_s_ref, wdown_s_ref, out_ref):
    e = pl.program_id(1)
    h = h_ref[...]
    scores = jnp.dot(h, wg_ref[...], preferred_element_type=_F32)
    ii = lax.broadcasted_iota(jnp.int32, (RC, NEXP), 1)
    m1 = jnp.max(scores, axis=-1, keepdims=True)
    a1 = jnp.min(jnp.where(scores == m1, ii, NEXP), axis=-1, keepdims=True)
    s2 = jnp.where(ii == a1, -jnp.inf, scores)
    m2 = jnp.max(s2, axis=-1, keepdims=True)
    a2 = jnp.min(jnp.where(s2 == m2, ii, NEXP), axis=-1, keepdims=True)
    e2 = jnp.exp(m2 - m1)
    w1 = 1.0 / (1.0 + e2)
    w2 = 1.0 - w1
    we = jnp.where(a1 == e, w1, 0.0) + jnp.where(a2 == e, w2, 0.0)

    gate = _silu(jnp.dot(h, wgate_s_ref[0], preferred_element_type=_F32))
    up = jnp.dot(h, wup_s_ref[0], preferred_element_type=_F32)
    ffn = jnp.dot(gate * up, wdown_s_ref[0], preferred_element_type=_F32)
    contrib = we * ffn

    @pl.when(e == 0)
    def _():
        out_ref[...] = h + contrib

    @pl.when(e > 0)
    def _():
        out_ref[...] = out_ref[...] + contrib


def _moe_block(h, mp):
    wg_t = mp['W_gate'].T
    wgate_s = jnp.stack([ep['Wg'].T for ep in mp['experts']])
    wup_s = jnp.stack([ep['Wu'].T for ep in mp['experts']])
    wdown_s = jnp.stack([ep['Wd'].T for ep in mp['experts']])

    return pl.pallas_call(
        _moe_body,
        grid=(L // RC, NEXP),
        in_specs=[
            pl.BlockSpec((RC, DIM), lambda r, e: (r, 0)),
            pl.BlockSpec((DIM, NEXP), lambda r, e: (0, 0)),
            pl.BlockSpec((1, DIM, FFI), lambda r, e: (e, 0, 0)),
            pl.BlockSpec((1, DIM, FFI), lambda r, e: (e, 0, 0)),
            pl.BlockSpec((1, FFI, DIM), lambda r, e: (e, 0, 0)),
        ],
        out_specs=pl.BlockSpec((RC, DIM), lambda r, e: (r, 0)),
        out_shape=jax.ShapeDtypeStruct((L, DIM), _F32),
    )(h, wg_t, wgate_s, wup_s, wdown_s)


def _head_body(h_ref, w_ref, out_ref):
    out_ref[...] = jax.nn.sigmoid(
        jnp.dot(h_ref[...], w_ref[...], preferred_element_type=_F32))


def _head(h, w_head):
    return pl.pallas_call(
        _head_body,
        grid=(L // RC,),
        in_specs=[
            pl.BlockSpec((RC, DIM), lambda r: (r, 0)),
            pl.BlockSpec((DIM, DIM), lambda r: (0, 0)),
        ],
        out_specs=pl.BlockSpec((RC, DIM), lambda r: (r, 0)),
        out_shape=jax.ShapeDtypeStruct((L, DIM), _F32),
    )(h, w_head.T)


def kernel(x, params):
    h = x.reshape(L, DIM)
    for i in range(len(params['blocks'])):
        h = _mamba_block(h, params['blocks'][i])
        h = _moe_block(h, params['moes'][i])
    h = _head(h, params['W_head'])
    return h.reshape(x.shape)


# speedup vs baseline: 10.9496x; 1.6728x over previous
import functools

import jax
import jax.numpy as jnp
from jax import lax
from jax.experimental import pallas as pl
from jax.experimental.pallas import tpu as pltpu

L = 2048
DIM = 1024
DIN = 2048
DSTATE = 16
DTRANK = 64
DCONV = 4
NEXP = 8
FFI = 2048
LC = 64
RC = 256

_F32 = jnp.float32
_NT = (((1,), (1,)), ((), ()))


def _silu(v):
    return v * jax.nn.sigmoid(v)


def _softplus(v):
    return jnp.maximum(v, 0.0) + jnp.log1p(jnp.exp(-jnp.abs(v)))


def _ntdot(a, b):
    return lax.dot_general(a, b, _NT, preferred_element_type=_F32)


def _mamba_body(x_ref, w_in_ref, conv_w_ref, conv_b_ref, wx_ref,
                w_dt_ref, b_dt_ref, alog_ref, dd_ref, w_out_ref,
                out_ref, tail_ref, state_ref, da_s, dbu_s, st_s):
    c = pl.program_id(0)

    @pl.when(c == 0)
    def _():
        tail_ref[...] = jnp.zeros_like(tail_ref)
        state_ref[...] = jnp.zeros_like(state_ref)

    xch = x_ref[...]
    xz = _ntdot(xch, w_in_ref[...])
    xc = xz[:, :DIN]
    res = xz[:, DIN:]

    ext = jnp.concatenate([tail_ref[...], xc], axis=0)
    tail_ref[...] = xc[LC - (DCONV - 1):, :]
    conv = conv_b_ref[...]
    for k in range(DCONV):
        conv = conv + ext[k:k + LC, :] * conv_w_ref[k:k + 1, :]
    xcs = _silu(conv)

    x_dbl = _ntdot(xcs, wx_ref[...])
    delta_r = x_dbl[:, :DTRANK]
    bm = x_dbl[:, DTRANK:DTRANK + DSTATE]
    cm = x_dbl[:, DTRANK + DSTATE:]
    delta = _softplus(_ntdot(delta_r, w_dt_ref[...]) + b_dt_ref[...])
    u = delta * xcs

    a2 = -jnp.exp(alog_ref[...])
    da_s[...] = jnp.exp(delta[:, None, :] * a2[None, :, :])
    dbu_s[...] = bm[:, :, None] * u[:, None, :]

    def step(l, _):
        st = (da_s[pl.ds(l, 1)][0] * state_ref[...]
              + dbu_s[pl.ds(l, 1)][0])
        state_ref[...] = st
        st_s[pl.ds(l, 1)] = st[None]
        return 0

    lax.fori_loop(0, LC, step, 0, unroll=False)

    y = jnp.sum(st_s[...] * cm[:, :, None], axis=1)
    y = y + xcs * dd_ref[...]
    y = y * _silu(res)
    out_ref[...] = _ntdot(y, w_out_ref[...]) + xch


def _mamba_block(h, bp):
    conv_w_t = bp['conv_w'].T
    conv_b = bp['conv_b'].reshape(1, DIN)
    b_dt = bp['b_dt'].reshape(1, DIN)
    alog_t = bp['A_log'].T
    dd = bp['D'].reshape(1, DIN)

    grid = L // LC
    full = lambda shape: pl.BlockSpec(shape, lambda c: (0,) * len(shape))
    return pl.pallas_call(
        _mamba_body,
        grid=(grid,),
        in_specs=[
            pl.BlockSpec((LC, DIM), lambda c: (c, 0)),
            full((2 * DIN, DIM)),
            full((DCONV, DIN)),
            full((1, DIN)),
            full((DTRANK + 2 * DSTATE, DIN)),
            full((DIN, DTRANK)),
            full((1, DIN)),
            full((DSTATE, DIN)),
            full((1, DIN)),
            full((DIM, DIN)),
        ],
        out_specs=pl.BlockSpec((LC, DIM), lambda c: (c, 0)),
        out_shape=jax.ShapeDtypeStruct((L, DIM), _F32),
        scratch_shapes=[
            pltpu.VMEM((DCONV - 1, DIN), _F32),
            pltpu.VMEM((DSTATE, DIN), _F32),
            pltpu.VMEM((LC, DSTATE, DIN), _F32),
            pltpu.VMEM((LC, DSTATE, DIN), _F32),
            pltpu.VMEM((LC, DSTATE, DIN), _F32),
        ],
    )(h, bp['W_in'], conv_w_t, conv_b, bp['W_x'], bp['W_dt'], b_dt,
      alog_t, dd, bp['W_out'])


def _top2_weight(h, wgate, e):
    scores = _ntdot(h, wgate)
    ii = lax.broadcasted_iota(jnp.int32, scores.shape, 1)
    m1 = jnp.max(scores, axis=-1, keepdims=True)
    a1 = jnp.min(jnp.where(scores == m1, ii, NEXP), axis=-1, keepdims=True)
    s2 = jnp.where(ii == a1, -jnp.inf, scores)
    m2 = jnp.max(s2, axis=-1, keepdims=True)
    a2 = jnp.min(jnp.where(s2 == m2, ii, NEXP), axis=-1, keepdims=True)
    e2 = jnp.exp(m2 - m1)
    w1 = 1.0 / (1.0 + e2)
    w2 = 1.0 - w1
    return jnp.where(a1 == e, w1, 0.0) + jnp.where(a2 == e, w2, 0.0)


def _moe_exp_body(e, h_ref, acc_ref, wgate_ref, wg_ref, wu_ref, wd_ref,
                  out_ref):
    h = h_ref[...]
    we = _top2_weight(h, wgate_ref[...], e)
    gate = _silu(_ntdot(h, wg_ref[...]))
    up = _ntdot(h, wu_ref[...])
    ffn = _ntdot(gate * up, wd_ref[...])
    out_ref[...] = acc_ref[...] + we * ffn


def _moe_block(h, mp):
    acc = h
    for e in range(NEXP):
        ep = mp['experts'][e]
        acc = pl.pallas_call(
            functools.partial(_moe_exp_body, e),
            grid=(L // RC,),
            in_specs=[
                pl.BlockSpec((RC, DIM), lambda r: (r, 0)),
                pl.BlockSpec((RC, DIM), lambda r: (r, 0)),
                pl.BlockSpec((NEXP, DIM), lambda r: (0, 0)),
                pl.BlockSpec((FFI, DIM), lambda r: (0, 0)),
                pl.BlockSpec((FFI, DIM), lambda r: (0, 0)),
                pl.BlockSpec((DIM, FFI), lambda r: (0, 0)),
            ],
            out_specs=pl.BlockSpec((RC, DIM), lambda r: (r, 0)),
            out_shape=jax.ShapeDtypeStruct((L, DIM), _F32),
        )(h, acc, mp['W_gate'], ep['Wg'], ep['Wu'], ep['Wd'])
    return acc


def _head_body(h_ref, w_ref, out_ref):
    out_ref[...] = jax.nn.sigmoid(_ntdot(h_ref[...], w_ref[...]))


def _head(h, w_head):
    return pl.pallas_call(
        _head_body,
        grid=(L // RC,),
        in_specs=[
            pl.BlockSpec((RC, DIM), lambda r: (r, 0)),
            pl.BlockSpec((DIM, DIM), lambda r: (0, 0)),
        ],
        out_specs=pl.BlockSpec((RC, DIM), lambda r: (r, 0)),
        out_shape=jax.ShapeDtypeStruct((L, DIM), _F32),
    )(h, w_head)


def kernel(x, params):
    h = x.reshape(L, DIM)
    for i in range(len(params['blocks'])):
        h = _mamba_block(h, params['blocks'][i])
        h = _moe_block(h, params['moes'][i])
    h = _head(h, params['W_head'])
    return h.reshape(x.shape)
